# baseline (speedup 1.0000x reference)
import functools

import jax
import jax.numpy as jnp
from jax import lax
from jax.experimental import pallas as pl
from jax.experimental.pallas import tpu as pltpu
from jax.experimental.pallas import tpu_sc as plsc

_TOPK = 20
_POWER = 0.1
_N = 256
_LANES = 16
_NCHUNK = _N // _LANES
_NCORES = 2
_NSUB = 16
_HI0 = 0x3F800000


def _sc_body(blk_hbm, pack_hbm, bits_v, pack_v):
    c = lax.axis_index("c")
    s = lax.axis_index("s")

    pltpu.sync_copy(blk_hbm.at[c, s], bits_v)

    def _ld(jb, jj):
        return bits_v[jb >> 2, ((jb & 3) << 1) + (jj >> 3),
                      pl.ds((jj & 7) * _LANES, _LANES)]

    def _bs_body(_, carry):
        lo, hi, clv = carry
        mid = (lo + hi) >> 1
        mid1 = mid + 1

        def _cnt_body(jb, negcnt):
            for jj in range(_LANES):
                negcnt = negcnt + ((_ld(jb, jj) - mid1) >> 31)
            return negcnt

        cnt = -lax.fori_loop(0, _NCHUNK, _cnt_body,
                             jnp.zeros((_LANES,), jnp.int32))
        ge = cnt >= _TOPK
        return (jnp.where(ge, lo, mid + 1),
                jnp.where(ge, mid, hi),
                jnp.where(ge, clv, cnt))

    tv, _, cl = lax.fori_loop(
        0, 30, _bs_body,
        (jnp.zeros((_LANES,), jnp.int32),
         jnp.full((_LANES,), _HI0, jnp.int32),
         jnp.zeros((_LANES,), jnp.int32)))
    need = _TOPK - cl

    def _sel_body(jb, eq_before):
        pack = jnp.zeros((_LANES,), jnp.int32)
        for jj in range(_LANES):
            u = _ld(jb, jj)
            lt = u < tv
            eq = u == tv
            sel = lt | (eq & (eq_before < need))
            pack = pack | (jnp.where(sel, 1, 0) << jj)
            eq_before = eq_before + jnp.where(eq, 1, 0)
        pack_v[jb >> 3, jb & 7] = pack
        return eq_before

    lax.fori_loop(0, _NCHUNK, _sel_body, jnp.zeros((_LANES,), jnp.int32))

    pltpu.sync_copy(
        pack_v,
        pack_hbm.at[c, :, s >> 3, :, pl.ds((s & 7) * _LANES, _LANES)])


_sc_topk_select = functools.partial(
    pl.kernel,
    out_type=jax.ShapeDtypeStruct((_NCORES, 2, 2, 8, 128), jnp.int32),
    mesh=plsc.VectorSubcoreMesh(core_axis_name="c", subcore_axis_name="s"),
    compiler_params=pltpu.CompilerParams(use_tc_tiling_on_sc=False),
    scratch_types=[
        pltpu.VMEM((4, 8, 128), jnp.int32),
        pltpu.VMEM((2, 8, _LANES), jnp.int32),
    ],
)(_sc_body)


def _tc_colsum_body(emb_ref, wrd_ref, out_ref):
    e = emb_ref[...]
    w = wrd_ref[...]
    f = 2.0 / (jnp.power(e, _POWER) + jnp.power(w, _POWER))
    out_ref[...] = jnp.sum(f, axis=0, keepdims=True)


def _tc_combine_body(colsum_ref, pack_ref, out_ref):
    colsum = colsum_ref[...]

    jrow = lax.broadcasted_iota(jnp.int32, (_N, _NCHUNK), 0)
    jcol = lax.broadcasted_iota(jnp.int32, (_N, _NCHUNK), 1)
    expand = jnp.where((jrow >> 4) == jcol, 1.0, 0.0)
    shifts = lax.broadcasted_iota(jnp.int32, (_N, _N), 0) & 15
    ones = jnp.ones((1, _N), jnp.float32)

    def _counts(p):
        rep = lax.dot_general(expand, p.astype(jnp.float32),
                              (((1,), (0,)), ((), ())),
                              preferred_element_type=jnp.float32)
        bits = ((rep.astype(jnp.int32) >> shifts) & 1).astype(jnp.float32)
        return lax.dot_general(ones, bits, (((1,), (1,)), ((), ())),
                               preferred_element_type=jnp.float32)

    def _as2d(mat):
        p = pack_ref[mat]
        return jnp.concatenate(
            [jnp.concatenate([p[0, 0], p[0, 1]], axis=1),
             jnp.concatenate([p[1, 0], p[1, 1]], axis=1)], axis=0)

    c_emb = _counts(_as2d(0))
    c_wrd = _counts(_as2d(1))
    m = jnp.where((c_emb > 0.0) & (c_wrd > 0.0), 1.0, 0.0)
    k = jnp.sum(m)
    out_ref[...] = jnp.sum(colsum * (256.0 * m + k), keepdims=True)


def kernel(emb_row, wrd_row):
    bits = jnp.stack([lax.bitcast_convert_type(emb_row, jnp.int32),
                      lax.bitcast_convert_type(wrd_row, jnp.int32)])
    blk = (bits.reshape(2, _NSUB, _LANES, _N).transpose(0, 1, 3, 2)
           .reshape(2, _NSUB, 4, 8, 128))
    pack = _sc_topk_select(blk)
    colsum = pl.pallas_call(
        _tc_colsum_body,
        out_shape=jax.ShapeDtypeStruct((1, _N), jnp.float32),
    )(emb_row, wrd_row)
    score = pl.pallas_call(
        _tc_combine_body,
        out_shape=jax.ShapeDtypeStruct((1, 1), jnp.float32),
    )(colsum, pack)
    return score[0, 0]


# speedup vs baseline: 1.0704x; 1.0704x over previous
import functools

import jax
import jax.numpy as jnp
from jax import lax
from jax.experimental import pallas as pl
from jax.experimental.pallas import tpu as pltpu
from jax.experimental.pallas import tpu_sc as plsc

_TOPK = 20
_POWER = 0.1
_N = 256
_LANES = 16
_NCHUNK = _N // _LANES
_NCORES = 2
_NSUB = 16
_HI0 = 0x3F800000


def _sc_body(bitsT_hbm, pack_hbm, bits_v, pack_v):
    c = lax.axis_index("c")
    s = lax.axis_index("s")

    pltpu.sync_copy(bitsT_hbm.at[:, pl.ds(c * _N + s * _LANES, _LANES)],
                    bits_v)

    def _bs_body(_, carry):
        lo, hi, clv = carry
        mid = (lo + hi) >> 1
        mid1 = mid + 1

        def _cnt_body(jb, negcnt):
            base = jb * _LANES
            for jj in range(_LANES):
                negcnt = negcnt + ((bits_v[base + jj] - mid1) >> 31)
            return negcnt

        cnt = -lax.fori_loop(0, _NCHUNK, _cnt_body,
                             jnp.zeros((_LANES,), jnp.int32))
        ge = cnt >= _TOPK
        return (jnp.where(ge, lo, mid + 1),
                jnp.where(ge, mid, hi),
                jnp.where(ge, clv, cnt))

    tv, _, cl = lax.fori_loop(
        0, 30, _bs_body,
        (jnp.zeros((_LANES,), jnp.int32),
         jnp.full((_LANES,), _HI0, jnp.int32),
         jnp.zeros((_LANES,), jnp.int32)))
    need = _TOPK - cl

    def _sel_body(jb, eq_before):
        base = jb * _LANES
        pack = jnp.zeros((_LANES,), jnp.int32)
        for jj in range(_LANES):
            u = bits_v[base + jj]
            lt = u < tv
            eq = u == tv
            sel = lt | (eq & (eq_before < need))
            pack = pack | (jnp.where(sel, 1, 0) << jj)
            eq_before = eq_before + jnp.where(eq, 1, 0)
        pack_v[jb >> 3, jb & 7] = pack
        return eq_before

    lax.fori_loop(0, _NCHUNK, _sel_body, jnp.zeros((_LANES,), jnp.int32))

    pltpu.sync_copy(
        pack_v,
        pack_hbm.at[c, :, s >> 3, :, pl.ds((s & 7) * _LANES, _LANES)])


_sc_topk_select = functools.partial(
    pl.kernel,
    out_type=jax.ShapeDtypeStruct((_NCORES, 2, 2, 8, 128), jnp.int32),
    mesh=plsc.VectorSubcoreMesh(core_axis_name="c", subcore_axis_name="s"),
    compiler_params=pltpu.CompilerParams(use_tc_tiling_on_sc=False),
    scratch_types=[
        pltpu.VMEM((_N, _LANES), jnp.int32),
        pltpu.VMEM((2, 8, _LANES), jnp.int32),
    ],
)(_sc_body)


def _tc_colsum_body(emb_ref, wrd_ref, out_ref):
    e = emb_ref[...]
    w = wrd_ref[...]
    f = 2.0 / (jnp.power(e, _POWER) + jnp.power(w, _POWER))
    out_ref[...] = jnp.sum(f, axis=0, keepdims=True)


def _tc_combine_body(colsum_ref, pack_ref, out_ref):
    colsum = colsum_ref[...]

    jrow = lax.broadcasted_iota(jnp.int32, (_N, _NCHUNK), 0)
    jcol = lax.broadcasted_iota(jnp.int32, (_N, _NCHUNK), 1)
    expand = jnp.where((jrow >> 4) == jcol, 1.0, 0.0)
    shifts = lax.broadcasted_iota(jnp.int32, (_N, _N), 0) & 15
    ones = jnp.ones((1, _N), jnp.float32)

    def _counts(p):
        rep = lax.dot_general(expand, p.astype(jnp.float32),
                              (((1,), (0,)), ((), ())),
                              preferred_element_type=jnp.float32)
        bits = ((rep.astype(jnp.int32) >> shifts) & 1).astype(jnp.float32)
        return lax.dot_general(ones, bits, (((1,), (1,)), ((), ())),
                               preferred_element_type=jnp.float32)

    def _as2d(mat):
        p = pack_ref[mat]
        return jnp.concatenate(
            [jnp.concatenate([p[0, 0], p[0, 1]], axis=1),
             jnp.concatenate([p[1, 0], p[1, 1]], axis=1)], axis=0)

    c_emb = _counts(_as2d(0))
    c_wrd = _counts(_as2d(1))
    m = jnp.where((c_emb > 0.0) & (c_wrd > 0.0), 1.0, 0.0)
    k = jnp.sum(m)
    out_ref[...] = jnp.sum(colsum * (256.0 * m + k), keepdims=True)


def kernel(emb_row, wrd_row):
    bits_t = jnp.concatenate(
        [lax.bitcast_convert_type(emb_row, jnp.int32).T,
         lax.bitcast_convert_type(wrd_row, jnp.int32).T], axis=1)
    pack = _sc_topk_select(bits_t)
    colsum = pl.pallas_call(
        _tc_colsum_body,
        out_shape=jax.ShapeDtypeStruct((1, _N), jnp.float32),
    )(emb_row, wrd_row)
    score = pl.pallas_call(
        _tc_combine_body,
        out_shape=jax.ShapeDtypeStruct((1, 1), jnp.float32),
    )(colsum, pack)
    return score[0, 0]
